# Initial kernel scaffold; baseline (speedup 1.0000x reference)
import functools

import jax
import jax.numpy as jnp
from jax import lax
from jax.experimental import pallas as pl
from jax.experimental.pallas import tpu as pltpu
from jax.experimental.pallas import tpu_sc as plsc

NG = 64
NG1 = NG + 1
CELL = NG1 * NG1
DXN = 8
DZN = 8
OUT = 128
BATCH = 4096
NPAIR = DXN * DZN
ROWS_PER_B = NPAIR * 4
HALF = 128
B_PER_TILE = 128
CHUNK = 32
LANES = 16


def _cdf_bin(v):
    e = jnp.exp(-jnp.abs(v))
    c = jnp.where(v > 0.0, 1.0 - 0.5 * e, 0.5 * e)
    s = c * float(NG)
    return jnp.clip(s.astype(jnp.int32), 0, NG - 1)


def _sc_body(num_cores, table, x_hbm, z_hbm, bord_hbm, invl_hbm, out_hbm,
             xv, zv, bordv, invlv, ixv, izv, dxv, dzv,
             idxb, wb, rows0, rows1, outb, sem0, sem1):
    wid = lax.axis_index("s") * num_cores + lax.axis_index("c")
    b0 = wid * B_PER_TILE

    pltpu.sync_copy(x_hbm.at[:, pl.ds(b0, B_PER_TILE)], xv)
    pltpu.sync_copy(z_hbm.at[:, pl.ds(b0, B_PER_TILE)], zv)
    pltpu.sync_copy(bord_hbm, bordv)
    pltpu.sync_copy(invl_hbm, invlv)

    lane = jnp.arange(LANES, dtype=jnp.int32)

    def fire(t, rows_ref, sem):
        idx_slice = idxb.at[pl.ds(t * HALF, HALF)]
        pltpu.make_async_copy(table.at[idx_slice], rows_ref, sem).start()

    def drain(rows_ref, sem):
        pltpu.make_async_copy(table.at[idxb.at[pl.ds(0, HALF)]],
                              rows_ref, sem).wait()

    def chunk_body(c, _):
        for g in range(CHUNK // LANES):
            off = c * CHUNK + g * LANES
            for src, dst_i, dst_d, n in ((xv, ixv, dxv, DXN),
                                         (zv, izv, dzv, DZN)):
                for i in range(n):
                    v = src[i, pl.ds(off, LANES)]
                    idx = _cdf_bin(v)
                    left = plsc.load_gather(bordv, [idx])
                    il = plsc.load_gather(invlv, [idx])
                    d = (v - left) * il
                    dst_i[i, pl.ds(g * LANES, LANES)] = idx
                    dst_d[i, pl.ds(g * LANES, LANES)] = d

        def build_b(b, _):
            lvec = jnp.full((LANES,), b, jnp.int32)
            for q in range(4):
                pairs = lane + q * LANES
                ixs = pairs >> 3
                izs = pairs & 7
                pbase = pairs * CELL
                i_x = plsc.load_gather(ixv, [ixs, lvec])
                i_z = plsc.load_gather(izv, [izs, lvec])
                dx = plsc.load_gather(dxv, [ixs, lvec])
                dz = plsc.load_gather(dzv, [izs, lvec])
                r00 = pbase + i_x * NG1 + i_z
                omdx = 1.0 - dx
                omdz = 1.0 - dz
                pos = b * ROWS_PER_B + q * LANES * 4 + lane
                plsc.store_scatter(idxb, [pos], r00)
                plsc.store_scatter(idxb, [pos + LANES], r00 + 1)
                plsc.store_scatter(idxb, [pos + 2 * LANES], r00 + NG1)
                plsc.store_scatter(idxb, [pos + 3 * LANES], r00 + NG1 + 1)
                plsc.store_scatter(wb, [pos], omdx * omdz)
                plsc.store_scatter(wb, [pos + LANES], omdx * dz)
                plsc.store_scatter(wb, [pos + 2 * LANES], dx * omdz)
                plsc.store_scatter(wb, [pos + 3 * LANES], dx * dz)
            return 0

        lax.fori_loop(0, CHUNK, build_b, 0)

        def acc_half(t, rows_ref, accs):
            def r_body(r, accs):
                w = plsc.load_gather(
                    wb, [jnp.full((LANES,), t * HALF + r, jnp.int32)])
                return tuple(
                    accs[k] + w * rows_ref[r, pl.ds(k * LANES, LANES)]
                    for k in range(OUT // LANES))
            return lax.fori_loop(0, HALF, r_body, accs)

        fire(0, rows0, sem0)

        def b_loop(b, _):
            accs = tuple(jnp.zeros((LANES,), jnp.float32)
                         for _ in range(OUT // LANES))
            t0 = 2 * b
            fire(t0 + 1, rows1, sem1)
            drain(rows0, sem0)
            accs = acc_half(t0, rows0, accs)

            @pl.when(t0 + 2 < 2 * CHUNK)
            def _():
                fire(t0 + 2, rows0, sem0)

            drain(rows1, sem1)
            accs = acc_half(t0 + 1, rows1, accs)
            for k in range(OUT // LANES):
                outb[b, pl.ds(k * LANES, LANES)] = accs[k]
            return 0

        lax.fori_loop(0, CHUNK, b_loop, 0)
        pltpu.sync_copy(outb, out_hbm.at[pl.ds(b0 + c * CHUNK, CHUNK), :])
        return 0

    lax.fori_loop(0, B_PER_TILE // CHUNK, chunk_body, 0)


@jax.jit
def _run(table, x, z, bord_p, invl):
    info = plsc.get_sparse_core_info()
    mesh = plsc.VectorSubcoreMesh(core_axis_name="c", subcore_axis_name="s")
    body = functools.partial(_sc_body, info.num_cores)
    kfn = pl.kernel(
        body,
        out_type=jax.ShapeDtypeStruct((BATCH, OUT), jnp.float32),
        mesh=mesh,
        scratch_types=[
            pltpu.VMEM((DXN, B_PER_TILE), jnp.float32),
            pltpu.VMEM((DZN, B_PER_TILE), jnp.float32),
            pltpu.VMEM((72,), jnp.float32),
            pltpu.VMEM((NG,), jnp.float32),
            pltpu.VMEM((DXN, CHUNK), jnp.int32),
            pltpu.VMEM((DZN, CHUNK), jnp.int32),
            pltpu.VMEM((DXN, CHUNK), jnp.float32),
            pltpu.VMEM((DZN, CHUNK), jnp.float32),
            pltpu.VMEM((CHUNK * ROWS_PER_B,), jnp.int32),
            pltpu.VMEM((CHUNK * ROWS_PER_B,), jnp.float32),
            pltpu.VMEM((HALF, OUT), jnp.float32),
            pltpu.VMEM((HALF, OUT), jnp.float32),
            pltpu.VMEM((CHUNK, OUT), jnp.float32),
            pltpu.SemaphoreType.DMA,
            pltpu.SemaphoreType.DMA,
        ],
    )
    return kfn(table, x, z, bord_p, invl)


def kernel(x, z, W, borders, inv_len):
    table = jnp.transpose(W, (3, 4, 0, 1, 2)).reshape(NPAIR * CELL, OUT)
    bord_p = jnp.concatenate([borders, jnp.zeros((7,), borders.dtype)])
    out_t = _run(table, x, z, bord_p, inv_len)
    return out_t.T


# speedup vs baseline: 5.1899x; 5.1899x over previous
import functools

import jax
import jax.numpy as jnp
from jax import lax
from jax.experimental import pallas as pl
from jax.experimental.pallas import tpu as pltpu
from jax.experimental.pallas import tpu_sc as plsc

NG = 64
NG1 = NG + 1
CELL = NG1 * NG1
DXN = 8
DZN = 8
OUT = 128
BATCH = 4096
NPAIR = DXN * DZN
ROWS_PER_B = NPAIR * 4
HALF = 128
B_PER_TILE = 128
CHUNK = 32
LANES = 16


def _cdf_bin(v):
    e = jnp.exp(-jnp.abs(v))
    c = jnp.where(v > 0.0, 1.0 - 0.5 * e, 0.5 * e)
    s = c * float(NG)
    return jnp.clip(s.astype(jnp.int32), 0, NG - 1)


def _sc_body(num_cores, table, x_hbm, z_hbm, bord_hbm, invl_hbm, out_hbm,
             xv, zv, bordv, invlv, ixv, izv, dxv, dzv,
             idxb, wb, rows0, rows1, outb, sem0, sem1):
    wid = lax.axis_index("s") * num_cores + lax.axis_index("c")
    b0 = wid * B_PER_TILE

    pltpu.sync_copy(x_hbm.at[:, pl.ds(b0, B_PER_TILE)], xv)
    pltpu.sync_copy(z_hbm.at[:, pl.ds(b0, B_PER_TILE)], zv)
    pltpu.sync_copy(bord_hbm, bordv)
    pltpu.sync_copy(invl_hbm, invlv)

    lane = jnp.arange(LANES, dtype=jnp.int32)

    def fire(t, rows_ref, sem):
        idx_slice = idxb.at[pl.ds(t * HALF, HALF)]
        pltpu.make_async_copy(table.at[idx_slice], rows_ref, sem).start()

    def drain(rows_ref, sem):
        pltpu.make_async_copy(table.at[idxb.at[pl.ds(0, HALF)]],
                              rows_ref, sem).wait()

    def chunk_body(c, _):
        for g in range(CHUNK // LANES):
            off = c * CHUNK + g * LANES
            for src, dst_i, dst_d, n in ((xv, ixv, dxv, DXN),
                                         (zv, izv, dzv, DZN)):
                for i in range(n):
                    v = src[i, pl.ds(off, LANES)]
                    idx = _cdf_bin(v)
                    left = plsc.load_gather(bordv, [idx])
                    il = plsc.load_gather(invlv, [idx])
                    d = (v - left) * il
                    dst_i[i, pl.ds(g * LANES, LANES)] = idx
                    dst_d[i, pl.ds(g * LANES, LANES)] = d

        def build_b(b, _):
            lvec = jnp.full((LANES,), b, jnp.int32)
            for q in range(4):
                pairs = lane + q * LANES
                ixs = pairs >> 3
                izs = pairs & 7
                pbase = pairs * CELL
                i_x = plsc.load_gather(ixv, [ixs, lvec])
                i_z = plsc.load_gather(izv, [izs, lvec])
                dx = plsc.load_gather(dxv, [ixs, lvec])
                dz = plsc.load_gather(dzv, [izs, lvec])
                r00 = pbase + i_x * NG1 + i_z
                omdx = 1.0 - dx
                omdz = 1.0 - dz
                pos = b * ROWS_PER_B + q * LANES * 4 + lane
                plsc.store_scatter(idxb, [pos], r00)
                plsc.store_scatter(idxb, [pos + LANES], r00 + 1)
                plsc.store_scatter(idxb, [pos + 2 * LANES], r00 + NG1)
                plsc.store_scatter(idxb, [pos + 3 * LANES], r00 + NG1 + 1)
                plsc.store_scatter(wb, [pos], omdx * omdz)
                plsc.store_scatter(wb, [pos + LANES], omdx * dz)
                plsc.store_scatter(wb, [pos + 2 * LANES], dx * omdz)
                plsc.store_scatter(wb, [pos + 3 * LANES], dx * dz)
            return 0

        lax.fori_loop(0, CHUNK, build_b, 0)

        def acc_half(t, rows_ref, accs):
            def r_body(r, accs):
                w = plsc.load_gather(
                    wb, [jnp.full((LANES,), t * HALF + r, jnp.int32)])
                return tuple(
                    accs[k] + w * rows_ref[r, pl.ds(k * LANES, LANES)]
                    for k in range(OUT // LANES))
            return lax.fori_loop(0, HALF, r_body, accs)

        fire(0, rows0, sem0)

        def b_loop(b, _):
            accs = tuple(jnp.zeros((LANES,), jnp.float32)
                         for _ in range(OUT // LANES))
            t0 = 2 * b
            fire(t0 + 1, rows1, sem1)
            drain(rows0, sem0)
            accs = acc_half(t0, rows0, accs)

            @pl.when(t0 + 2 < 2 * CHUNK)
            def _():
                fire(t0 + 2, rows0, sem0)

            drain(rows1, sem1)
            accs = acc_half(t0 + 1, rows1, accs)
            for k in range(OUT // LANES):
                outb[b, pl.ds(k * LANES, LANES)] = accs[k]
            return 0

        lax.fori_loop(0, CHUNK, b_loop, 0)
        pltpu.sync_copy(outb, out_hbm.at[pl.ds(b0 + c * CHUNK, CHUNK), :])
        return 0

    lax.fori_loop(0, B_PER_TILE // CHUNK, chunk_body, 0)


@jax.jit
def _run(table, x, z, bord_p, invl):
    info = plsc.get_sparse_core_info()
    mesh = plsc.VectorSubcoreMesh(core_axis_name="c", subcore_axis_name="s")
    body = functools.partial(_sc_body, info.num_cores)
    kfn = pl.kernel(
        body,
        out_type=jax.ShapeDtypeStruct((BATCH, OUT), jnp.float32),
        mesh=mesh,
        scratch_types=[
            pltpu.VMEM((DXN, B_PER_TILE), jnp.float32),
            pltpu.VMEM((DZN, B_PER_TILE), jnp.float32),
            pltpu.VMEM((72,), jnp.float32),
            pltpu.VMEM((NG,), jnp.float32),
            pltpu.VMEM((DXN, CHUNK), jnp.int32),
            pltpu.VMEM((DZN, CHUNK), jnp.int32),
            pltpu.VMEM((DXN, CHUNK), jnp.float32),
            pltpu.VMEM((DZN, CHUNK), jnp.float32),
            pltpu.VMEM((CHUNK * ROWS_PER_B,), jnp.int32),
            pltpu.VMEM((CHUNK * ROWS_PER_B,), jnp.float32),
            pltpu.VMEM((HALF, OUT), jnp.float32),
            pltpu.VMEM((HALF, OUT), jnp.float32),
            pltpu.VMEM((CHUNK, OUT), jnp.float32),
            pltpu.SemaphoreType.DMA,
            pltpu.SemaphoreType.DMA,
        ],
        compiler_params=pltpu.CompilerParams(needs_layout_passes=False),
    )
    return kfn(table, x, z, bord_p, invl)


def kernel(x, z, W, borders, inv_len):
    table = jnp.transpose(W, (3, 4, 0, 1, 2)).reshape(NPAIR * CELL, OUT)
    bord_p = jnp.concatenate([borders, jnp.zeros((7,), borders.dtype)])
    out_t = _run(table, x, z, bord_p, inv_len)
    return out_t.T
